# baseline (speedup 1.0000x reference)
import functools

import jax
import jax.numpy as jnp
from jax import lax
from jax.experimental import pallas as pl
from jax.experimental.pallas import tpu as pltpu
from jax.experimental.pallas import tpu_sc as plsc

_N = 100000
_E = 6400000
_NB = 5
_RMIN, _RMAX = 0.5, 6.0
_BETA = _NB**2 / _RMAX**2
_DSTEP = (_RMAX - _RMIN) / _NB

_NC, _NS = 2, 16
_NW = _NC * _NS
_EPW = _E // _NW
_C = 400
_NCHUNK = _EPW // _C
_GPC = _C // 16
_NP = 100352
_RPT = _NP // _NS

_mesh = plsc.VectorSubcoreMesh(
    core_axis_name="c", subcore_axis_name="s", num_cores=_NC, num_subcores=_NS
)


def _dg(x, idx):
    return lax.gather(
        x,
        idx.reshape(16, 1),
        lax.GatherDimensionNumbers(
            offset_dims=(), collapsed_slice_dims=(0,), start_index_map=(0,)
        ),
        (1,),
        mode=lax.GatherScatterMode.PROMISE_IN_BOUNDS,
    )


@functools.partial(
    pl.kernel,
    mesh=_mesh,
    out_type=jax.ShapeDtypeStruct((_NC, _NP, 16), jnp.float32),
    compiler_params=pltpu.CompilerParams(use_tc_tiling_on_sc=False),
    scratch_types=[
        pltpu.VMEM_SHARED((_NP, 16), jnp.float32),
        [pltpu.VMEM((_C,), jnp.int32) for _ in range(2)],
        [pltpu.VMEM((_C,), jnp.int32) for _ in range(4)],
        [[pltpu.VMEM((_C,), jnp.float32) for _ in range(6)] for _ in range(2)],
        [pltpu.VMEM((_C, 16), jnp.float32) for _ in range(2)],
        [pltpu.SemaphoreType.DMA for _ in range(2)],
        [pltpu.SemaphoreType.DMA for _ in range(2)],
    ],
)
def _rbf_sc(xh, yh, zh, i0h, i1h, zrh, out, acc, i0b, i1b, comp, ebuf, gsems, ssems):
    cid = lax.axis_index("c")
    sid = lax.axis_index("s")
    wid = cid * _NS + sid
    r0 = sid * _RPT

    pltpu.sync_copy(zrh, acc.at[pl.ds(r0, _RPT)])
    plsc.subcore_barrier()

    iot = lax.broadcasted_iota(jnp.int32, (16,), 0)
    shiftv = jnp.where(
        (iot & 7) < _NB,
        _RMIN + _DSTEP * (iot & 7).astype(jnp.float32),
        jnp.float32(1e6),
    )
    idx_hi = iot >> 3
    idx_swap = iot ^ 8

    def _issue(b, jm, ch):
        base = wid * _EPW + ch * _C
        pltpu.sync_copy(i0h.at[pl.ds(base, _C)], i0b[b])
        pltpu.sync_copy(i1h.at[pl.ds(base, _C)], i1b[jm])
        pltpu.async_copy(xh.at[i0b[b]], comp[b][0], gsems[b])
        pltpu.async_copy(yh.at[i0b[b]], comp[b][1], gsems[b])
        pltpu.async_copy(zh.at[i0b[b]], comp[b][2], gsems[b])
        pltpu.async_copy(xh.at[i1b[jm]], comp[b][3], gsems[b])
        pltpu.async_copy(yh.at[i1b[jm]], comp[b][4], gsems[b])
        pltpu.async_copy(zh.at[i1b[jm]], comp[b][5], gsems[b])

    def _compute(b):
        def _group(g, _):
            sl = pl.ds(g * 16, 16)
            dx = comp[b][0][sl] - comp[b][3][sl]
            dy = comp[b][1][sl] - comp[b][4][sl]
            dz = comp[b][2][sl] - comp[b][5][sl]
            d2 = dx * dx + dy * dy + dz * dz
            bits = lax.bitcast_convert_type(d2, jnp.int32)
            y = lax.bitcast_convert_type(
                jnp.int32(0x5F3759DF) - (bits >> 1), jnp.float32
            )
            for _i in range(3):
                y = y * (1.5 - 0.5 * d2 * y * y)
            dr = d2 * y
            for p in range(8):
                drb = _dg(dr, idx_hi + 2 * p)
                t = shiftv - drb
                v = jnp.exp((-_BETA) * (t * t))
                ebuf[b][g * 16 + 2 * p, :] = v
                ebuf[b][g * 16 + 2 * p + 1, :] = _dg(v, idx_swap)
            return 0

        lax.fori_loop(0, _GPC, _group, 0)

    def _drain_gathers(b):
        for j in range(6):
            pltpu.make_async_copy(
                xh.at[pl.ds(0, _C)], comp[b][j], gsems[b]
            ).wait()

    def _wait_scatter(b):
        pltpu.make_async_copy(ebuf[b], acc.at[i1b[0]], ssems[b]).wait()

    def _phase(j, i):
        b = j % 2
        ch = 4 * i + j
        _drain_gathers(b)

        @pl.when(ch >= 2)
        def _():
            _wait_scatter(b)

        @pl.when(ch + 1 < _NCHUNK)
        def _():
            _issue(1 - b, (j + 1) % 4, ch + 1)

        _compute(b)
        pltpu.async_copy(ebuf[b], acc.at[i1b[j]], ssems[b], add=True)

    _issue(0, 0, 0)

    def _quad(i, _):
        for j in range(4):
            _phase(j, i)
        return 0

    lax.fori_loop(0, _NCHUNK // 4, _quad, 0)
    _wait_scatter(0)
    _wait_scatter(1)

    plsc.subcore_barrier()
    pltpu.sync_copy(acc.at[pl.ds(r0, _RPT)], out.at[cid, pl.ds(r0, _RPT)])


def _add_body(p_ref, o_ref):
    o_ref[...] = p_ref[0] + p_ref[1]


def kernel(R, neighbor_idx):
    R = R.astype(jnp.float32)
    x, y, z = R[:, 0], R[:, 1], R[:, 2]
    i0 = neighbor_idx[0].astype(jnp.int32)
    i1 = neighbor_idx[1].astype(jnp.int32)
    zrows = jnp.zeros((_RPT, 16), jnp.float32)
    partial = _rbf_sc(x, y, z, i0, i1, zrows)
    p2 = partial.reshape(_NC, _NP * 16 // 128, 128)
    summed = pl.pallas_call(
        _add_body,
        out_shape=jax.ShapeDtypeStruct((_NP * 16 // 128, 128), jnp.float32),
    )(p2)
    return summed.reshape(_NP, 16)[:_N, :_NB]


# speedup vs baseline: 1.5207x; 1.5207x over previous
import functools

import jax
import jax.numpy as jnp
from jax import lax
from jax.experimental import pallas as pl
from jax.experimental.pallas import tpu as pltpu
from jax.experimental.pallas import tpu_sc as plsc

_N = 100000
_E = 6400000
_NB = 5
_RMIN, _RMAX = 0.5, 6.0
_BETA = _NB**2 / _RMAX**2
_DSTEP = (_RMAX - _RMIN) / _NB

_NC, _NS = 2, 16
_NW = _NC * _NS
_EPW = _E // _NW
_C = 2000
_NCHUNK = _EPW // _C
_GPC = _C // 16
_NP = 100352
_RPT = _NP // _NS

_mesh = plsc.VectorSubcoreMesh(
    core_axis_name="c", subcore_axis_name="s", num_cores=_NC, num_subcores=_NS
)


def _dg(x, idx):
    return lax.gather(
        x,
        idx.reshape(16, 1),
        lax.GatherDimensionNumbers(
            offset_dims=(), collapsed_slice_dims=(0,), start_index_map=(0,)
        ),
        (1,),
        mode=lax.GatherScatterMode.PROMISE_IN_BOUNDS,
    )


@functools.partial(
    pl.kernel,
    mesh=_mesh,
    out_type=jax.ShapeDtypeStruct((_NC, _NB, _NP), jnp.float32),
    compiler_params=pltpu.CompilerParams(use_tc_tiling_on_sc=False),
    scratch_types=[
        [pltpu.VMEM_SHARED((_NP,), jnp.float32) for _ in range(_NB)],
        [pltpu.VMEM((_C,), jnp.int32) for _ in range(2)],
        [pltpu.VMEM((_C,), jnp.int32) for _ in range(4)],
        [[pltpu.VMEM((_C,), jnp.float32) for _ in range(6)] for _ in range(2)],
        [[pltpu.VMEM((_C,), jnp.float32) for _ in range(_NB)] for _ in range(2)],
        [pltpu.SemaphoreType.DMA for _ in range(2)],
        [pltpu.SemaphoreType.DMA for _ in range(2)],
        [pltpu.SemaphoreType.DMA for _ in range(2)],
    ],
)
def _rbf_sc(xh, yh, zh, i0h, i1h, zrh, out, acc, i0b, i1b, comp, ebuf, gsems, ssems, isems):
    cid = lax.axis_index("c")
    sid = lax.axis_index("s")
    wid = cid * _NS + sid
    r0 = sid * _RPT

    for k in range(_NB):
        pltpu.sync_copy(zrh, acc[k].at[pl.ds(r0, _RPT)])
    plsc.subcore_barrier()

    iot = lax.broadcasted_iota(jnp.int32, (16,), 0)
    shifts = [_RMIN + _DSTEP * k for k in range(_NB)]

    def _prefetch_idx(b, jm, ch):
        base = wid * _EPW + ch * _C
        pltpu.async_copy(i0h.at[pl.ds(base, _C)], i0b[b], isems[b])
        pltpu.async_copy(i1h.at[pl.ds(base, _C)], i1b[jm], isems[b])

    def _wait_idx(b):
        pltpu.make_async_copy(i0h.at[pl.ds(0, _C)], i0b[b], isems[b]).wait()
        pltpu.make_async_copy(i0h.at[pl.ds(0, _C)], i1b[0], isems[b]).wait()

    def _issue(b, jm, ch):
        pltpu.async_copy(xh.at[i0b[b]], comp[b][0], gsems[b])
        pltpu.async_copy(yh.at[i0b[b]], comp[b][1], gsems[b])
        pltpu.async_copy(zh.at[i0b[b]], comp[b][2], gsems[b])
        pltpu.async_copy(xh.at[i1b[jm]], comp[b][3], gsems[b])
        pltpu.async_copy(yh.at[i1b[jm]], comp[b][4], gsems[b])
        pltpu.async_copy(zh.at[i1b[jm]], comp[b][5], gsems[b])

    def _compute(b):
        def _group(g, _):
            sl = pl.ds(g * 16, 16)
            dx = comp[b][0][sl] - comp[b][3][sl]
            dy = comp[b][1][sl] - comp[b][4][sl]
            dz = comp[b][2][sl] - comp[b][5][sl]
            d2 = dx * dx + dy * dy + dz * dz
            bits = lax.bitcast_convert_type(d2, jnp.int32)
            y = lax.bitcast_convert_type(
                jnp.int32(0x5F3759DF) - (bits >> 1), jnp.float32
            )
            for _i in range(3):
                y = y * (1.5 - 0.5 * d2 * y * y)
            dr = d2 * y
            for k in range(_NB):
                t = shifts[k] - dr
                ebuf[b][k][sl] = jnp.exp((-_BETA) * (t * t))
            return 0

        lax.fori_loop(0, _GPC, _group, 0)

    def _drain_gathers(b):
        for j in range(6):
            pltpu.make_async_copy(
                xh.at[pl.ds(0, _C)], comp[b][j], gsems[b]
            ).wait()

    def _wait_scatter(b):
        for k in range(_NB):
            pltpu.make_async_copy(
                ebuf[b][k], acc[k].at[i1b[0]], ssems[b]
            ).wait()

    def _phase(j, i):
        b = j % 2
        ch = 4 * i + j
        _drain_gathers(b)

        @pl.when(ch >= 2)
        def _():
            _wait_scatter(b)

        @pl.when(ch + 1 < _NCHUNK)
        def _():
            _wait_idx(1 - b)
            _issue(1 - b, (j + 1) % 4, ch + 1)

        @pl.when(ch + 2 < _NCHUNK)
        def _():
            _prefetch_idx(b, (j + 2) % 4, ch + 2)

        _compute(b)
        for k in range(_NB):
            pltpu.async_copy(ebuf[b][k], acc[k].at[i1b[j]], ssems[b], add=True)

    base0 = wid * _EPW
    pltpu.sync_copy(i0h.at[pl.ds(base0, _C)], i0b[0])
    pltpu.sync_copy(i1h.at[pl.ds(base0, _C)], i1b[0])
    _prefetch_idx(1, 1, 1)
    _issue(0, 0, 0)

    def _quad(i, _):
        for j in range(4):
            _phase(j, i)
        return 0

    lax.fori_loop(0, _NCHUNK // 4, _quad, 0)
    _wait_scatter(0)
    _wait_scatter(1)

    plsc.subcore_barrier()
    for k in range(_NB):
        pltpu.sync_copy(
            acc[k].at[pl.ds(r0, _RPT)], out.at[cid, k, pl.ds(r0, _RPT)]
        )


def _add_body(p_ref, o_ref):
    o_ref[...] = p_ref[0] + p_ref[1]


def kernel(R, neighbor_idx):
    R = R.astype(jnp.float32)
    x, y, z = R[:, 0], R[:, 1], R[:, 2]
    i0 = neighbor_idx[0].astype(jnp.int32)
    i1 = neighbor_idx[1].astype(jnp.int32)
    zrows = jnp.zeros((_RPT,), jnp.float32)
    partial = _rbf_sc(x, y, z, i0, i1, zrows)
    p2 = partial.reshape(_NC, _NB * _NP // 128, 128)
    summed = pl.pallas_call(
        _add_body,
        out_shape=jax.ShapeDtypeStruct((_NB * _NP // 128, 128), jnp.float32),
    )(p2)
    return summed.reshape(_NB, _NP)[:, :_N].T
